# baseline (speedup 1.0000x reference)
import functools

import jax
import jax.numpy as jnp
from jax import lax
from jax.experimental import pallas as pl
from jax.experimental.pallas import tpu as pltpu
from jax.experimental.pallas import tpu_sc as plsc

_D = 256
_K = 8192
_BETA = 0.25
_MB = 1024
_NW = 32
_CHUNK = 96


_SUB = 64
_LN = 128


def _argmin_body(r_ref, w_ref, n2r_ref, n2w_ref, lane_ref, idx_ref, minv_ref):
    mm2 = lax.dot_general(
        r_ref[...], w_ref[...], (((1,), (1,)), ((), ())),
        preferred_element_type=jnp.float32)
    lane = lane_ref[...]
    for mi in range(_MB // _SUB):
        base = mi * _SUB
        n2rb = jnp.broadcast_to(n2r_ref[pl.ds(base, _SUB), :], (_SUB, _LN))
        accv = ((n2rb + jnp.broadcast_to(n2w_ref[:, 0:_LN], (_SUB, _LN)))
                - mm2[base:base + _SUB, 0:_LN])
        acci = jnp.zeros((_SUB, _LN), jnp.float32)
        for j in range(1, _K // _LN):
            t = n2rb + jnp.broadcast_to(
                n2w_ref[:, j * _LN:(j + 1) * _LN], (_SUB, _LN))
            d = t - mm2[base:base + _SUB, j * _LN:(j + 1) * _LN]
            c = d < accv
            accv = jnp.where(c, d, accv)
            acci = jnp.where(c, jnp.float32(j), acci)
        m = jnp.min(accv, axis=1, keepdims=True)
        gidx = acci * jnp.float32(_LN) + lane
        idxf = jnp.min(jnp.where(accv == m, gidx, jnp.float32(_K)),
                       axis=1, keepdims=True)
        idx_ref[pl.ds(base, _SUB), :] = idxf.astype(jnp.int32)
        minv_ref[pl.ds(base, _SUB), :] = m


def _tc_argmin(r, W, n2r, n2w):
    n = r.shape[0]
    lanef = jnp.arange(_LN, dtype=jnp.float32).reshape(1, _LN)
    r2b = (r * 2.0).astype(jnp.bfloat16)
    wb = W.astype(jnp.bfloat16)
    return pl.pallas_call(
        _argmin_body,
        grid=(n // _MB,),
        in_specs=[
            pl.BlockSpec((_MB, _D), lambda i: (i, 0)),
            pl.BlockSpec((_K, _D), lambda i: (0, 0)),
            pl.BlockSpec((_MB, 1), lambda i: (i, 0)),
            pl.BlockSpec((1, _K), lambda i: (0, 0)),
            pl.BlockSpec((1, _LN), lambda i: (0, 0)),
        ],
        out_specs=[
            pl.BlockSpec((_MB, 1), lambda i: (i, 0)),
            pl.BlockSpec((_MB, 1), lambda i: (i, 0)),
        ],
        out_shape=[
            jax.ShapeDtypeStruct((n, 1), jnp.int32),
            jax.ShapeDtypeStruct((n, 1), jnp.float32),
        ],
    )(r2b, wb, n2r, n2w, lanef)


@functools.cache
def _sc_gather(n):
    b_per_w = n // _NW
    nchunk = b_per_w // _CHUNK
    mesh = plsc.VectorSubcoreMesh(core_axis_name="c", subcore_axis_name="s")

    @functools.partial(
        pl.kernel, mesh=mesh,
        out_type=jax.ShapeDtypeStruct((n, _D), jnp.float32),
        scratch_types=[
            pltpu.VMEM((nchunk, _CHUNK), jnp.int32),
            pltpu.VMEM((b_per_w, _D), jnp.float32),
            pltpu.SemaphoreType.DMA,
        ],
    )
    def gather(table_hbm, idx_hbm, out_hbm, idx_v, rows_v, sem):
        wid = lax.axis_index("s") * 2 + lax.axis_index("c")
        pltpu.sync_copy(idx_hbm.at[wid], idx_v)
        copies = [
            pltpu.async_copy(
                table_hbm.at[idx_v.at[j]],
                rows_v.at[pl.ds(j * _CHUNK, _CHUNK)], sem)
            for j in range(nchunk)
        ]
        for c in copies:
            c.wait()
        pltpu.sync_copy(rows_v, out_hbm.at[pl.ds(wid * b_per_w, b_per_w)])

    return gather


def kernel(latents, W):
    B, CD = latents.shape
    n = B * (CD // _D)
    r = jnp.reshape(latents, (n, _D))
    n2r = jnp.sum(r ** 2, axis=1, keepdims=True)
    n2w = jnp.sum(W ** 2, axis=1)
    idx, minv = _tc_argmin(r, W, n2r, jnp.reshape(n2w, (1, _K)))
    idx3 = jnp.reshape(idx, (_NW, (n // _NW) // _CHUNK, _CHUNK))
    q = _sc_gather(n)(W, idx3)
    quantized_st = jnp.reshape(q, (B, CD))
    vq_loss = ((1.0 + _BETA) / B) * jnp.sum(minv)
    return quantized_st, vq_loss


# speedup vs baseline: 1.1063x; 1.1063x over previous
import functools

import jax
import jax.numpy as jnp
from jax import lax
from jax.experimental import pallas as pl
from jax.experimental.pallas import tpu as pltpu
from jax.experimental.pallas import tpu_sc as plsc

_D = 256
_K = 8192
_BETA = 0.25
_MB = 1024
_NW = 32
_CHUNK = 96


_SUB = 64
_LN = 128


def _argmin_body(r_ref, w_ref, n2r_ref, n2w_ref, lane_ref, idx_ref, minv_ref):
    mm2 = lax.dot_general(
        r_ref[...] * 2.0, w_ref[...], (((1,), (1,)), ((), ())),
        preferred_element_type=jnp.float32)
    lane = lane_ref[...]
    for mi in range(_MB // _SUB):
        base = mi * _SUB
        n2rb = jnp.broadcast_to(n2r_ref[pl.ds(base, _SUB), :], (_SUB, _LN))
        accv = ((n2rb + jnp.broadcast_to(n2w_ref[:, 0:_LN], (_SUB, _LN)))
                - mm2[base:base + _SUB, 0:_LN])
        acci = jnp.zeros((_SUB, _LN), jnp.float32)
        for j in range(1, _K // _LN):
            t = n2rb + jnp.broadcast_to(
                n2w_ref[:, j * _LN:(j + 1) * _LN], (_SUB, _LN))
            d = t - mm2[base:base + _SUB, j * _LN:(j + 1) * _LN]
            c = d < accv
            accv = jnp.where(c, d, accv)
            acci = jnp.where(c, jnp.float32(j), acci)
        m = jnp.min(accv, axis=1, keepdims=True)
        gidx = acci * jnp.float32(_LN) + lane
        idxf = jnp.min(jnp.where(accv == m, gidx, jnp.float32(_K)),
                       axis=1, keepdims=True)
        idx_ref[pl.ds(base, _SUB), :] = idxf.astype(jnp.int32)
        minv_ref[pl.ds(base, _SUB), :] = m


def _tc_argmin(r, W, n2r, n2w):
    n = r.shape[0]
    lanef = jnp.arange(_LN, dtype=jnp.float32).reshape(1, _LN)
    return pl.pallas_call(
        _argmin_body,
        grid=(n // _MB,),
        in_specs=[
            pl.BlockSpec((_MB, _D), lambda i: (i, 0)),
            pl.BlockSpec((_K, _D), lambda i: (0, 0)),
            pl.BlockSpec((_MB, 1), lambda i: (i, 0)),
            pl.BlockSpec((1, _K), lambda i: (0, 0)),
            pl.BlockSpec((1, _LN), lambda i: (0, 0)),
        ],
        out_specs=[
            pl.BlockSpec((_MB, 1), lambda i: (i, 0)),
            pl.BlockSpec((_MB, 1), lambda i: (i, 0)),
        ],
        out_shape=[
            jax.ShapeDtypeStruct((n, 1), jnp.int32),
            jax.ShapeDtypeStruct((n, 1), jnp.float32),
        ],
    )(r, W, n2r, n2w, lanef)


@functools.cache
def _sc_gather(n):
    b_per_w = n // _NW
    nchunk = b_per_w // _CHUNK
    mesh = plsc.VectorSubcoreMesh(core_axis_name="c", subcore_axis_name="s")

    @functools.partial(
        pl.kernel, mesh=mesh,
        out_type=jax.ShapeDtypeStruct((n, _D), jnp.float32),
        scratch_types=[
            pltpu.VMEM((nchunk, _CHUNK), jnp.int32),
            pltpu.VMEM((b_per_w, _D), jnp.float32),
            pltpu.SemaphoreType.DMA,
        ],
    )
    def gather(table_hbm, idx_hbm, out_hbm, idx_v, rows_v, sem):
        wid = lax.axis_index("s") * 2 + lax.axis_index("c")
        pltpu.sync_copy(idx_hbm.at[wid], idx_v)
        copies = [
            pltpu.async_copy(
                table_hbm.at[idx_v.at[j]],
                rows_v.at[pl.ds(j * _CHUNK, _CHUNK)], sem)
            for j in range(nchunk)
        ]
        for c in copies:
            c.wait()
        pltpu.sync_copy(rows_v, out_hbm.at[pl.ds(wid * b_per_w, b_per_w)])

    return gather


def kernel(latents, W):
    B, CD = latents.shape
    n = B * (CD // _D)
    r = jnp.reshape(latents, (n, _D))
    n2r = jnp.sum(r ** 2, axis=1, keepdims=True)
    n2w = jnp.sum(W ** 2, axis=1)
    idx, minv = _tc_argmin(r, W, n2r, jnp.reshape(n2w, (1, _K)))
    idx3 = jnp.reshape(idx, (_NW, (n // _NW) // _CHUNK, _CHUNK))
    q = _sc_gather(n)(W, idx3)
    quantized_st = jnp.reshape(q, (B, CD))
    vq_loss = ((1.0 + _BETA) / B) * jnp.sum(minv)
    return quantized_st, vq_loss
